# baseline (speedup 1.0000x reference)
import functools

import jax
import jax.numpy as jnp
from jax import lax
from jax.experimental import pallas as pl
from jax.experimental.pallas import tpu as pltpu
from jax.experimental.pallas import tpu_sc as plsc

NNZ = 41
BATCH = 64
NC = 2
NS = 16
NROWS = 4096
RPG = 2
GIDX = RPG * NNZ
LANES = 16
CB = BATCH // NC
CHUNKS = CB // LANES
RPS = NROWS // NS
GPS = RPS // RPG
VCHUNK_OFF = (0, 16, 32, 48, 64, 66)


def _wchunk(j):
    if j < 80:
        return j // 16, j % 16
    return 5, j - 66


_mesh = plsc.VectorSubcoreMesh(core_axis_name="c", subcore_axis_name="s")


@functools.partial(
    pl.kernel,
    mesh=_mesh,
    out_type=jax.ShapeDtypeStruct((NROWS, BATCH), jnp.float32),
    compiler_params=pltpu.CompilerParams(use_tc_tiling_on_sc=False),
    scratch_types=[
        pltpu.VMEM_SHARED((NROWS, CB), jnp.float32),
        pltpu.VMEM_SHARED((NROWS, CB), jnp.float32),
        pltpu.VMEM((GPS, GIDX), jnp.int32),
        pltpu.VMEM((GPS, GIDX), jnp.float32),
        pltpu.VMEM((GPS, GIDX), jnp.int32),
        pltpu.VMEM((GPS, GIDX), jnp.float32),
        pltpu.VMEM((GIDX, CB), jnp.float32),
        pltpu.VMEM((GIDX, CB), jnp.float32),
        pltpu.VMEM((RPS, CB), jnp.float32),
        pltpu.SemaphoreType.DMA,
        pltpu.SemaphoreType.DMA,
    ],
)
def _fused(x, bcols, bvals, acols, avals, out, x_s, t_s,
           bcols_v, bvals_v, acols_v, avals_v, buf0, buf1, out_v,
           sem0, sem1):
    cid = lax.axis_index("c")
    sid = lax.axis_index("s")
    c0 = cid * CB
    r0 = sid * RPS
    g0 = sid * GPS

    pltpu.sync_copy(x.at[pl.ds(r0, RPS), pl.ds(c0, CB)],
                    x_s.at[pl.ds(r0, RPS)])
    pltpu.sync_copy(bcols.at[pl.ds(g0, GPS)], bcols_v)
    pltpu.sync_copy(bvals.at[pl.ds(g0, GPS)], bvals_v)
    pltpu.sync_copy(acols.at[pl.ds(g0, GPS)], acols_v)
    pltpu.sync_copy(avals.at[pl.ds(g0, GPS)], avals_v)
    plsc.subcore_barrier()

    bufs = (buf0, buf1)
    sems = (sem0, sem1)

    def run_stage(tab_s, cols_v, vals_v):
        pltpu.make_async_copy(tab_s.at[cols_v.at[0]], buf0, sem0).start()

        def body(i, carry):
            for b in range(2):
                g = 2 * i + b
                buf, sem = bufs[b], sems[b]
                pltpu.make_async_copy(tab_s.at[cols_v.at[g]], buf,
                                      sem).wait()

                nxt = g + 1

                @pl.when(nxt < GPS)
                def _():
                    pltpu.make_async_copy(
                        tab_s.at[cols_v.at[nxt]], bufs[1 - b], sems[1 - b]
                    ).start()

                for r in range(RPG):
                    acc = [jnp.zeros((LANES,), jnp.float32)
                           for _ in range(CHUNKS)]
                    vv = [vals_v[g, pl.ds(off, LANES)]
                          for off in VCHUNK_OFF]
                    for j in range(NNZ):
                        e = r * NNZ + j
                        ck, lane = _wchunk(e)
                        v = vv[ck][lane]
                        for c in range(CHUNKS):
                            acc[c] = acc[c] + v * buf[e, pl.ds(c * LANES,
                                                               LANES)]
                    for c in range(CHUNKS):
                        out_v[RPG * g + r, pl.ds(c * LANES, LANES)] = acc[c]
            return carry

        lax.fori_loop(0, GPS // 2, body, 0)

    run_stage(x_s, bcols_v, bvals_v)
    pltpu.sync_copy(out_v, t_s.at[pl.ds(r0, RPS)])
    plsc.subcore_barrier()

    run_stage(t_s, acols_v, avals_v)
    pltpu.sync_copy(out_v, out.at[pl.ds(r0, RPS), pl.ds(c0, CB)])


def kernel(x, a_row_ids, a_cols, a_vals, b_row_ids, b_cols, b_vals):
    groups = NROWS // RPG
    bc = b_cols.reshape(groups, GIDX)
    bv = b_vals.reshape(groups, GIDX)
    ac = a_cols.reshape(groups, GIDX)
    av = a_vals.reshape(groups, GIDX)
    y = _fused(x, bc, bv, ac, av)
    return jnp.transpose(y)[None, :, :]
